# Initial kernel scaffold; baseline (speedup 1.0000x reference)
import jax
import jax.numpy as jnp
from jax.experimental import pallas as pl


def kernel(x, edge_index, batch, W1, b1, g1, be1, rm1, rv1, W2, b2, g2, be2, rm2, rv2, Wc1, bc1, Wc2, bc2, Wc3, bc3, Wc4, bc4, Wc5, bc5, Wfc, bfc):
    raise NotImplementedError("write your pallas kernel here")


# speedup vs baseline: 21.9353x; 21.9353x over previous
import functools

import jax
import jax.numpy as jnp
from jax import lax
from jax.experimental import pallas as pl
from jax.experimental.pallas import tpu as pltpu
from jax.experimental.pallas import tpu_sc as plsc

NN = 10000
NNP = 10240
NE = 640000
NCORE = 2
NSUB = 16
NW = NCORE * NSUB
KCH = 80
NCHUNK = NE // (NW * KCH)
RPT = NNP // NSUB


def _sc_mesh():
    return plsc.VectorSubcoreMesh(core_axis_name="c", subcore_axis_name="s")


def _prop_call(dp):
    @functools.partial(
        pl.kernel,
        mesh=_sc_mesh(),
        compiler_params=pltpu.CompilerParams(use_tc_tiling_on_sc=False),
        out_type=jax.ShapeDtypeStruct((NCORE, NNP, dp), jnp.float32),
        scratch_types=[
            pltpu.VMEM((NCHUNK, KCH), jnp.int32),
            pltpu.VMEM((NCHUNK, KCH), jnp.int32),
            pltpu.VMEM((KCH, dp), jnp.float32),
            pltpu.VMEM_SHARED((NNP, dp), jnp.float32),
            pltpu.SemaphoreType.DMA,
        ],
    )
    def prop(src_hbm, dst_hbm, zs_hbm, out_hbm, srcv, dstv, rows, acc, gsem):
        cid = lax.axis_index("c")
        sid = lax.axis_index("s")
        wid = sid * NCORE + cid
        pltpu.sync_copy(src_hbm.at[wid], srcv)
        pltpu.sync_copy(dst_hbm.at[wid], dstv)
        base = sid * RPT
        pltpu.sync_copy(zs_hbm.at[pl.ds(base, RPT)], acc.at[pl.ds(base, RPT)])
        plsc.subcore_barrier()

        def body(j, carry):
            pltpu.async_copy(zs_hbm.at[srcv.at[j]], rows, gsem).wait()
            pltpu.sync_copy(rows, acc.at[dstv.at[j]], add=True)
            return carry

        lax.fori_loop(0, NCHUNK, body, 0)
        plsc.subcore_barrier()
        pltpu.sync_copy(acc.at[pl.ds(base, RPT)],
                        out_hbm.at[cid].at[pl.ds(base, RPT)])

    return prop


def _deg_call():
    @functools.partial(
        pl.kernel,
        mesh=_sc_mesh(),
        compiler_params=pltpu.CompilerParams(use_tc_tiling_on_sc=False),
        out_type=jax.ShapeDtypeStruct((NCORE, NNP, 16), jnp.float32),
        scratch_types=[
            pltpu.VMEM((NCHUNK, KCH), jnp.int32),
            pltpu.VMEM((KCH, 16), jnp.float32),
            pltpu.VMEM((128, 16), jnp.float32),
            pltpu.VMEM_SHARED((NNP, 16), jnp.float32),
        ],
    )
    def degk(dst_hbm, out_hbm, dstv, ones_v, zv, acc):
        cid = lax.axis_index("c")
        sid = lax.axis_index("s")
        wid = sid * NCORE + cid
        pltpu.sync_copy(dst_hbm.at[wid], dstv)

        def fill_z(i, carry):
            zv[i, :] = jnp.zeros((16,), jnp.float32)
            return carry

        lax.fori_loop(0, 128, fill_z, 0)

        def fill_o(i, carry):
            ones_v[i, :] = jnp.ones((16,), jnp.float32)
            return carry

        lax.fori_loop(0, KCH, fill_o, 0)
        base = sid * RPT
        for m in range(5):
            pltpu.sync_copy(zv, acc.at[pl.ds(base + m * 128, 128)])
        plsc.subcore_barrier()

        def body(j, carry):
            pltpu.sync_copy(ones_v, acc.at[dstv.at[j]], add=True)
            return carry

        lax.fori_loop(0, NCHUNK, body, 0)
        plsc.subcore_barrier()
        pltpu.sync_copy(acc.at[pl.ds(base, RPT)],
                        out_hbm.at[cid].at[pl.ds(base, RPT)])

    return degk


def _ffn(xp, W1f, c1, W2f, c2):
    R = 512

    def body(x_ref, w1_ref, c1_ref, w2_ref, c2_ref, o_ref):
        h = jnp.dot(x_ref[...], w1_ref[...], preferred_element_type=jnp.float32)
        h = jnp.maximum(h + c1_ref[...], 0.0)
        o_ref[...] = jnp.dot(h, w2_ref[...],
                             preferred_element_type=jnp.float32) + c2_ref[...]

    return pl.pallas_call(
        body,
        grid=(NNP // R,),
        in_specs=[
            pl.BlockSpec((R, 128), lambda i: (i, 0)),
            pl.BlockSpec((128, 3840), lambda i: (0, 0)),
            pl.BlockSpec((1, 3840), lambda i: (0, 0)),
            pl.BlockSpec((3840, 128), lambda i: (0, 0)),
            pl.BlockSpec((1, 128), lambda i: (0, 0)),
        ],
        out_specs=pl.BlockSpec((R, 128), lambda i: (i, 0)),
        out_shape=jax.ShapeDtypeStruct((NNP, 128), jnp.float32),
    )(xp, W1f, c1, W2f, c2)


def _layer1(cnt, h0p, Wc1p):
    R = 1024

    def body(c_ref, h_ref, w_ref, zs_ref, dinv_ref):
        deg = c_ref[0, :, 0:1] + c_ref[1, :, 0:1] + 1.0
        dinv = lax.rsqrt(deg)
        z = jnp.dot(h_ref[...], w_ref[...], preferred_element_type=jnp.float32)
        zs_ref[...] = z * dinv
        dinv_ref[...] = dinv

    return pl.pallas_call(
        body,
        grid=(NNP // R,),
        in_specs=[
            pl.BlockSpec((2, R, 16), lambda i: (0, i, 0)),
            pl.BlockSpec((R, 128), lambda i: (i, 0)),
            pl.BlockSpec((128, 64), lambda i: (0, 0)),
        ],
        out_specs=[
            pl.BlockSpec((R, 64), lambda i: (i, 0)),
            pl.BlockSpec((R, 1), lambda i: (i, 0)),
        ],
        out_shape=[
            jax.ShapeDtypeStruct((NNP, 64), jnp.float32),
            jax.ShapeDtypeStruct((NNP, 1), jnp.float32),
        ],
    )(cnt, h0p, Wc1p)


def _mid(agg, zs_prev, dinv, Wc, bprev):
    R = 1024
    dpi = zs_prev.shape[1]
    dpo = Wc.shape[1]

    def body(a_ref, z_ref, d_ref, w_ref, b_ref, o_ref):
        s = a_ref[0] + a_ref[1] - z_ref[...]
        h = jnp.maximum(s * d_ref[...] + b_ref[...], 0.0)
        o_ref[...] = jnp.dot(h, w_ref[...],
                             preferred_element_type=jnp.float32) * d_ref[...]

    return pl.pallas_call(
        body,
        grid=(NNP // R,),
        in_specs=[
            pl.BlockSpec((2, R, dpi), lambda i: (0, i, 0)),
            pl.BlockSpec((R, dpi), lambda i: (i, 0)),
            pl.BlockSpec((R, 1), lambda i: (i, 0)),
            pl.BlockSpec((dpi, dpo), lambda i: (0, 0)),
            pl.BlockSpec((1, dpi), lambda i: (0, 0)),
        ],
        out_specs=pl.BlockSpec((R, dpo), lambda i: (i, 0)),
        out_shape=jax.ShapeDtypeStruct((NNP, dpo), jnp.float32),
    )(agg, zs_prev, dinv, Wc, bprev)


def _final_h(agg, zs_prev, dinv, bprev):
    R = 1024
    dpi = zs_prev.shape[1]

    def body(a_ref, z_ref, d_ref, b_ref, o_ref):
        s = a_ref[0] + a_ref[1] - z_ref[...]
        h = jnp.maximum(s * d_ref[...] + b_ref[...], 0.0)
        o_ref[...] = h[:, :4]

    return pl.pallas_call(
        body,
        grid=(NNP // R,),
        in_specs=[
            pl.BlockSpec((2, R, dpi), lambda i: (0, i, 0)),
            pl.BlockSpec((R, dpi), lambda i: (i, 0)),
            pl.BlockSpec((R, 1), lambda i: (i, 0)),
            pl.BlockSpec((1, dpi), lambda i: (0, 0)),
        ],
        out_specs=pl.BlockSpec((R, 4), lambda i: (i, 0)),
        out_shape=jax.ShapeDtypeStruct((NNP, 4), jnp.float32),
    )(agg, zs_prev, dinv, bprev)


def _head(h5flat, Wfc, bfc2):
    def body(h_ref, w_ref, b_ref, o_ref):
        o_ref[...] = jnp.dot(h_ref[...], w_ref[...],
                             preferred_element_type=jnp.float32) + b_ref[...]

    return pl.pallas_call(
        body,
        out_shape=jax.ShapeDtypeStruct((1, 10), jnp.float32),
    )(h5flat, Wfc, bfc2)


def kernel(x, edge_index, batch, W1, b1, g1, be1, rm1, rv1, W2, b2, g2, be2,
           rm2, rv2, Wc1, bc1, Wc2, bc2, Wc3, bc3, Wc4, bc4, Wc5, bc5, Wfc,
           bfc):
    eps = 1e-5
    xp = jnp.pad(x, ((0, NNP - NN), (0, 28)))
    s1 = g1 / jnp.sqrt(rv1 + eps)
    t1 = be1 - rm1 * s1
    W1f = jnp.pad(W1, ((0, 28), (0, 0))) * s1[None, :]
    c1 = (b1 * s1 + t1)[None, :]
    s2 = g2 / jnp.sqrt(rv2 + eps)
    t2 = be2 - rm2 * s2
    W2f = jnp.pad(W2 * s2[None, :], ((0, 0), (0, 28)))
    c2 = jnp.pad(b2 * s2 + t2, (0, 28))[None, :]
    Wc1p = jnp.pad(Wc1, ((0, 28), (0, 0)))
    Wc4p = jnp.pad(Wc4, ((0, 0), (0, 8)))
    Wc5p = jnp.pad(Wc5, ((0, 8), (0, 12)))
    bc4p = jnp.pad(bc4, (0, 8))[None, :]
    bc5p = jnp.pad(bc5, (0, 12))[None, :]
    src3 = edge_index[0].reshape(NW, NCHUNK, KCH)
    dst3 = edge_index[1].reshape(NW, NCHUNK, KCH)

    cnt = _deg_call()(dst3)
    h0p = _ffn(xp, W1f, c1, W2f, c2)

    zs1, dinv = _layer1(cnt, h0p, Wc1p)
    agg1 = _prop_call(64)(src3, dst3, zs1)
    zs2 = _mid(agg1, zs1, dinv, Wc2, bc1[None, :])
    agg2 = _prop_call(32)(src3, dst3, zs2)
    zs3 = _mid(agg2, zs2, dinv, Wc3, bc2[None, :])
    agg3 = _prop_call(16)(src3, dst3, zs3)
    zs4 = _mid(agg3, zs3, dinv, Wc4p, bc3[None, :])
    agg4 = _prop_call(16)(src3, dst3, zs4)
    zs5 = _mid(agg4, zs4, dinv, Wc5p, bc4p)
    agg5 = _prop_call(16)(src3, dst3, zs5)
    h5 = _final_h(agg5, zs5, dinv, bc5p)
    out = _head(h5[:NN].reshape(1, NN * 4), Wfc, bfc[None, :])
    return out


# speedup vs baseline: 49.2586x; 2.2456x over previous
import functools

import jax
import jax.numpy as jnp
from jax import lax
from jax.experimental import pallas as pl
from jax.experimental.pallas import tpu as pltpu
from jax.experimental.pallas import tpu_sc as plsc

NN = 10000
NNP = 10240
NE = 640000
NCORE = 2
NSUB = 16
NW = NCORE * NSUB
KCH = 80
NCHUNK = NE // (NW * KCH)
G = 5
NG = NCHUNK // G
RPT = NNP // NSUB


def _sc_mesh():
    return plsc.VectorSubcoreMesh(core_axis_name="c", subcore_axis_name="s")


def _prop_call(dp):
    @functools.partial(
        pl.kernel,
        mesh=_sc_mesh(),
        compiler_params=pltpu.CompilerParams(use_tc_tiling_on_sc=False),
        out_type=jax.ShapeDtypeStruct((NCORE, NNP, dp), jnp.float32),
        scratch_types=[
            pltpu.VMEM((NCHUNK, KCH), jnp.int32),
            pltpu.VMEM((NCHUNK, KCH), jnp.int32),
            pltpu.VMEM((2, G * KCH, dp), jnp.float32),
            pltpu.VMEM_SHARED((NNP, dp), jnp.float32),
            pltpu.SemaphoreType.DMA,
            pltpu.SemaphoreType.DMA,
        ],
    )
    def prop(src_hbm, dst_hbm, zs_hbm, out_hbm, srcv, dstv, rows, acc, gsem,
             ssem):
        cid = lax.axis_index("c")
        sid = lax.axis_index("s")
        wid = sid * NCORE + cid
        pltpu.sync_copy(src_hbm.at[wid], srcv)
        pltpu.sync_copy(dst_hbm.at[wid], dstv)
        base = sid * RPT
        pltpu.sync_copy(zs_hbm.at[pl.ds(base, RPT)], acc.at[pl.ds(base, RPT)])
        plsc.subcore_barrier()

        def fire_g(g, b):
            for k in range(G):
                pltpu.async_copy(zs_hbm.at[srcv.at[g * G + k]],
                                 rows.at[b].at[pl.ds(k * KCH, KCH)], gsem)

        def wait_g(b):
            for k in range(G):
                pltpu.make_async_copy(
                    zs_hbm.at[srcv.at[k]],
                    rows.at[b].at[pl.ds(k * KCH, KCH)], gsem).wait()

        def fire_s(g, b):
            for k in range(G):
                pltpu.async_copy(rows.at[b].at[pl.ds(k * KCH, KCH)],
                                 acc.at[dstv.at[g * G + k]], ssem, add=True)

        def wait_s(b):
            for k in range(G):
                pltpu.make_async_copy(
                    rows.at[b].at[pl.ds(k * KCH, KCH)],
                    acc.at[dstv.at[k]], ssem).wait()

        fire_g(0, 0)
        fire_g(1, 1)
        wait_g(0)
        fire_s(0, 0)

        def pair(i, carry):
            ga = 2 * i + 1
            gb = 2 * i + 2
            wait_s(0)
            fire_g(gb, 0)
            wait_g(1)
            fire_s(ga, 1)
            wait_s(1)
            fire_g(gb + 1, 1)
            wait_g(0)
            fire_s(gb, 0)
            return carry

        lax.fori_loop(0, (NG - 2) // 2, pair, 0)
        wait_s(0)
        wait_g(1)
        fire_s(NG - 1, 1)
        wait_s(1)
        plsc.subcore_barrier()
        pltpu.sync_copy(acc.at[pl.ds(base, RPT)],
                        out_hbm.at[cid].at[pl.ds(base, RPT)])

    return prop


def _deg_call():
    @functools.partial(
        pl.kernel,
        mesh=_sc_mesh(),
        compiler_params=pltpu.CompilerParams(use_tc_tiling_on_sc=False),
        out_type=jax.ShapeDtypeStruct((NCORE, NNP, 8), jnp.float32),
        scratch_types=[
            pltpu.VMEM((NCHUNK, KCH), jnp.int32),
            pltpu.VMEM((KCH, 8), jnp.float32),
            pltpu.VMEM_SHARED((NNP, 8), jnp.float32),
            pltpu.SemaphoreType.DMA,
        ],
    )
    def degk(dst_hbm, zeros_hbm, ones_hbm, out_hbm, dstv, ones_v, acc, ssem):
        cid = lax.axis_index("c")
        sid = lax.axis_index("s")
        wid = sid * NCORE + cid
        pltpu.sync_copy(dst_hbm.at[wid], dstv)
        pltpu.sync_copy(ones_hbm, ones_v)
        base = sid * RPT
        pltpu.sync_copy(zeros_hbm, acc.at[pl.ds(base, RPT)])
        plsc.subcore_barrier()

        for k in range(8):
            pltpu.async_copy(ones_v, acc.at[dstv.at[k]], ssem, add=True)

        def body(j, carry):
            pltpu.async_copy(ones_v, acc.at[dstv.at[j + 8]], ssem, add=True)
            pltpu.make_async_copy(ones_v, acc.at[dstv.at[j]], ssem).wait()
            return carry

        lax.fori_loop(0, NCHUNK - 8, body, 0)
        for k in range(8):
            pltpu.make_async_copy(ones_v, acc.at[dstv.at[k]], ssem).wait()
        plsc.subcore_barrier()
        pltpu.sync_copy(acc.at[pl.ds(base, RPT)],
                        out_hbm.at[cid].at[pl.ds(base, RPT)])

    return degk


def _ffn(xp, W1f, c1, W2f, c2):
    R = 512

    def body(x_ref, w1_ref, c1_ref, w2_ref, c2_ref, o_ref):
        h = jnp.dot(x_ref[...], w1_ref[...], preferred_element_type=jnp.float32)
        h = jnp.maximum(h + c1_ref[...], 0.0)
        o_ref[...] = jnp.dot(h, w2_ref[...],
                             preferred_element_type=jnp.float32) + c2_ref[...]

    return pl.pallas_call(
        body,
        grid=(NNP // R,),
        in_specs=[
            pl.BlockSpec((R, 128), lambda i: (i, 0)),
            pl.BlockSpec((128, 3840), lambda i: (0, 0)),
            pl.BlockSpec((1, 3840), lambda i: (0, 0)),
            pl.BlockSpec((3840, 128), lambda i: (0, 0)),
            pl.BlockSpec((1, 128), lambda i: (0, 0)),
        ],
        out_specs=pl.BlockSpec((R, 128), lambda i: (i, 0)),
        out_shape=jax.ShapeDtypeStruct((NNP, 128), jnp.float32),
    )(xp, W1f, c1, W2f, c2)


def _layer1(cnt, h0p, Wc1p):
    R = 1024

    def body(c_ref, h_ref, w_ref, zs_ref, dinv_ref):
        deg = c_ref[0, :, 0:1] + c_ref[1, :, 0:1] + 1.0
        dinv = lax.rsqrt(deg)
        z = jnp.dot(h_ref[...], w_ref[...], preferred_element_type=jnp.float32)
        zs_ref[...] = z * dinv
        dinv_ref[...] = dinv

    return pl.pallas_call(
        body,
        grid=(NNP // R,),
        in_specs=[
            pl.BlockSpec((2, R, 8), lambda i: (0, i, 0)),
            pl.BlockSpec((R, 128), lambda i: (i, 0)),
            pl.BlockSpec((128, 64), lambda i: (0, 0)),
        ],
        out_specs=[
            pl.BlockSpec((R, 64), lambda i: (i, 0)),
            pl.BlockSpec((R, 1), lambda i: (i, 0)),
        ],
        out_shape=[
            jax.ShapeDtypeStruct((NNP, 64), jnp.float32),
            jax.ShapeDtypeStruct((NNP, 1), jnp.float32),
        ],
    )(cnt, h0p, Wc1p)


def _mid(agg, zs_prev, dinv, Wc, bprev):
    R = 1024
    dpi = zs_prev.shape[1]
    dpo = Wc.shape[1]

    def body(a_ref, z_ref, d_ref, w_ref, b_ref, o_ref):
        s = a_ref[0] + a_ref[1] - z_ref[...]
        h = jnp.maximum(s * d_ref[...] + b_ref[...], 0.0)
        o_ref[...] = jnp.dot(h, w_ref[...],
                             preferred_element_type=jnp.float32) * d_ref[...]

    return pl.pallas_call(
        body,
        grid=(NNP // R,),
        in_specs=[
            pl.BlockSpec((2, R, dpi), lambda i: (0, i, 0)),
            pl.BlockSpec((R, dpi), lambda i: (i, 0)),
            pl.BlockSpec((R, 1), lambda i: (i, 0)),
            pl.BlockSpec((dpi, dpo), lambda i: (0, 0)),
            pl.BlockSpec((1, dpi), lambda i: (0, 0)),
        ],
        out_specs=pl.BlockSpec((R, dpo), lambda i: (i, 0)),
        out_shape=jax.ShapeDtypeStruct((NNP, dpo), jnp.float32),
    )(agg, zs_prev, dinv, Wc, bprev)


def _final_h(agg, zs_prev, dinv, bprev):
    R = 1024
    dpi = zs_prev.shape[1]

    def body(a_ref, z_ref, d_ref, b_ref, o_ref):
        s = a_ref[0] + a_ref[1] - z_ref[...]
        h = jnp.maximum(s * d_ref[...] + b_ref[...], 0.0)
        o_ref[...] = h[:, :4]

    return pl.pallas_call(
        body,
        grid=(NNP // R,),
        in_specs=[
            pl.BlockSpec((2, R, dpi), lambda i: (0, i, 0)),
            pl.BlockSpec((R, dpi), lambda i: (i, 0)),
            pl.BlockSpec((R, 1), lambda i: (i, 0)),
            pl.BlockSpec((1, dpi), lambda i: (0, 0)),
        ],
        out_specs=pl.BlockSpec((R, 4), lambda i: (i, 0)),
        out_shape=jax.ShapeDtypeStruct((NNP, 4), jnp.float32),
    )(agg, zs_prev, dinv, bprev)


def _head(h5flat, Wfc, bfc2):
    def body(h_ref, w_ref, b_ref, o_ref):
        o_ref[...] = jnp.dot(h_ref[...], w_ref[...],
                             preferred_element_type=jnp.float32) + b_ref[...]

    return pl.pallas_call(
        body,
        out_shape=jax.ShapeDtypeStruct((1, 10), jnp.float32),
    )(h5flat, Wfc, bfc2)


def kernel(x, edge_index, batch, W1, b1, g1, be1, rm1, rv1, W2, b2, g2, be2,
           rm2, rv2, Wc1, bc1, Wc2, bc2, Wc3, bc3, Wc4, bc4, Wc5, bc5, Wfc,
           bfc):
    eps = 1e-5
    xp = jnp.pad(x, ((0, NNP - NN), (0, 28)))
    s1 = g1 / jnp.sqrt(rv1 + eps)
    t1 = be1 - rm1 * s1
    W1f = jnp.pad(W1, ((0, 28), (0, 0))) * s1[None, :]
    c1 = (b1 * s1 + t1)[None, :]
    s2 = g2 / jnp.sqrt(rv2 + eps)
    t2 = be2 - rm2 * s2
    W2f = jnp.pad(W2 * s2[None, :], ((0, 0), (0, 28)))
    c2 = jnp.pad(b2 * s2 + t2, (0, 28))[None, :]
    Wc1p = jnp.pad(Wc1, ((0, 28), (0, 0)))
    Wc4p = jnp.pad(Wc4, ((0, 0), (0, 8)))
    Wc5p = jnp.pad(Wc5, ((0, 8), (0, 12)))
    bc4p = jnp.pad(bc4, (0, 8))[None, :]
    bc5p = jnp.pad(bc5, (0, 12))[None, :]
    src3 = edge_index[0].reshape(NW, NCHUNK, KCH)
    dst3 = edge_index[1].reshape(NW, NCHUNK, KCH)

    cnt = _deg_call()(dst3, jnp.zeros((RPT, 8), jnp.float32),
                      jnp.ones((KCH, 8), jnp.float32))
    h0p = _ffn(xp, W1f, c1, W2f, c2)

    zs1, dinv = _layer1(cnt, h0p, Wc1p)
    prop32 = _prop_call(32)
    agg1a = prop32(src3, dst3, zs1[:, :32])
    agg1b = prop32(src3, dst3, zs1[:, 32:])
    agg1 = jnp.concatenate([agg1a, agg1b], axis=2)
    zs2 = _mid(agg1, zs1, dinv, Wc2, bc1[None, :])
    agg2 = _prop_call(32)(src3, dst3, zs2)
    zs3 = _mid(agg2, zs2, dinv, Wc3, bc2[None, :])
    agg3 = _prop_call(16)(src3, dst3, zs3)
    zs4 = _mid(agg3, zs3, dinv, Wc4p, bc3[None, :])
    agg4 = _prop_call(16)(src3, dst3, zs4)
    zs5 = _mid(agg4, zs4, dinv, Wc5p, bc4p)
    agg5 = _prop_call(16)(src3, dst3, zs5)
    h5 = _final_h(agg5, zs5, dinv, bc5p)
    out = _head(h5[:NN].reshape(1, NN * 4), Wfc, bfc[None, :])
    return out


# speedup vs baseline: 55.1812x; 1.1202x over previous
import functools

import jax
import jax.numpy as jnp
from jax import lax
from jax.experimental import pallas as pl
from jax.experimental.pallas import tpu as pltpu
from jax.experimental.pallas import tpu_sc as plsc

NN = 10000
NNP = 10240
NE = 640000
NEP = 655360
NCORE = 2
NSUB = 16
NW = NCORE * NSUB
KCH = 128
NCHUNK = NEP // (NW * KCH)
G = 8
NG = NCHUNK // G
RPT = NNP // NSUB


def _sc_mesh():
    return plsc.VectorSubcoreMesh(core_axis_name="c", subcore_axis_name="s")


def _prop_call(dp):
    @functools.partial(
        pl.kernel,
        mesh=_sc_mesh(),
        compiler_params=pltpu.CompilerParams(use_tc_tiling_on_sc=False),
        out_type=jax.ShapeDtypeStruct((NCORE, NNP, dp), jnp.float32),
        scratch_types=[
            pltpu.VMEM((NCHUNK, KCH), jnp.int32),
            pltpu.VMEM((NCHUNK, KCH), jnp.int32),
            pltpu.VMEM((2, G * KCH, dp), jnp.float32),
            pltpu.VMEM_SHARED((NNP, dp), jnp.float32),
            pltpu.SemaphoreType.DMA,
            pltpu.SemaphoreType.DMA,
        ],
    )
    def prop(src_hbm, dst_hbm, zs_hbm, out_hbm, srcv, dstv, rows, acc, gsem,
             ssem):
        cid = lax.axis_index("c")
        sid = lax.axis_index("s")
        wid = sid * NCORE + cid
        base = sid * RPT
        c1 = pltpu.async_copy(src_hbm.at[wid], srcv, gsem)
        c2 = pltpu.async_copy(dst_hbm.at[wid], dstv, gsem)
        c3 = pltpu.async_copy(zs_hbm.at[pl.ds(base, RPT)],
                              acc.at[pl.ds(base, RPT)], ssem)
        c1.wait()
        c2.wait()
        c3.wait()
        plsc.subcore_barrier()

        def fire_g(g, b):
            for k in range(G):
                pltpu.async_copy(zs_hbm.at[srcv.at[g * G + k]],
                                 rows.at[b].at[pl.ds(k * KCH, KCH)], gsem)

        def wait_g(b):
            pltpu.make_async_copy(zs_hbm.at[pl.ds(0, G * KCH)],
                                  rows.at[b], gsem).wait()

        def fire_s(g, b):
            for k in range(G):
                pltpu.async_copy(rows.at[b].at[pl.ds(k * KCH, KCH)],
                                 acc.at[dstv.at[g * G + k]], ssem, add=True)

        def wait_s(b):
            pltpu.make_async_copy(rows.at[b],
                                  acc.at[pl.ds(0, G * KCH)], ssem).wait()

        fire_g(0, 0)
        fire_g(1, 1)
        wait_g(0)
        fire_s(0, 0)

        def pair(i, carry):
            ga = 2 * i + 1
            gb = 2 * i + 2
            wait_s(0)
            fire_g(gb, 0)
            wait_g(1)
            fire_s(ga, 1)
            wait_s(1)
            fire_g(gb + 1, 1)
            wait_g(0)
            fire_s(gb, 0)
            return carry

        lax.fori_loop(0, (NG - 2) // 2, pair, 0)
        wait_s(0)
        wait_g(1)
        fire_s(NG - 1, 1)
        wait_s(1)
        plsc.subcore_barrier()
        pltpu.sync_copy(acc.at[pl.ds(base, RPT)],
                        out_hbm.at[cid].at[pl.ds(base, RPT)])

    return prop


def _prop2_call():
    dp = 32

    @functools.partial(
        pl.kernel,
        mesh=_sc_mesh(),
        compiler_params=pltpu.CompilerParams(use_tc_tiling_on_sc=False),
        out_type=(jax.ShapeDtypeStruct((NCORE, NNP, dp), jnp.float32),
                  jax.ShapeDtypeStruct((NCORE, NNP, dp), jnp.float32)),
        scratch_types=[
            pltpu.VMEM((NCHUNK, KCH), jnp.int32),
            pltpu.VMEM((NCHUNK, KCH), jnp.int32),
            pltpu.VMEM((2, G * KCH, dp), jnp.float32),
            pltpu.VMEM_SHARED((NNP, dp), jnp.float32),
            pltpu.SemaphoreType.DMA,
            pltpu.SemaphoreType.DMA,
        ],
    )
    def prop2(src_hbm, dst_hbm, zsa_hbm, zsb_hbm, outa_hbm, outb_hbm,
              srcv, dstv, rows, acc, gsem, ssem):
        cid = lax.axis_index("c")
        sid = lax.axis_index("s")
        wid = sid * NCORE + cid
        base = sid * RPT
        c1 = pltpu.async_copy(src_hbm.at[wid], srcv, gsem)
        c2 = pltpu.async_copy(dst_hbm.at[wid], dstv, gsem)
        c1.wait()
        c2.wait()

        def fire_g(zs_hbm, g, b):
            for k in range(G):
                pltpu.async_copy(zs_hbm.at[srcv.at[g * G + k]],
                                 rows.at[b].at[pl.ds(k * KCH, KCH)], gsem)

        def wait_g(zs_hbm, b):
            pltpu.make_async_copy(zs_hbm.at[pl.ds(0, G * KCH)],
                                  rows.at[b], gsem).wait()

        def fire_s(g, b):
            for k in range(G):
                pltpu.async_copy(rows.at[b].at[pl.ds(k * KCH, KCH)],
                                 acc.at[dstv.at[g * G + k]], ssem, add=True)

        def wait_s(b):
            pltpu.make_async_copy(rows.at[b],
                                  acc.at[pl.ds(0, G * KCH)], ssem).wait()

        def phase(zs_hbm, out_hbm):
            pltpu.sync_copy(zs_hbm.at[pl.ds(base, RPT)],
                            acc.at[pl.ds(base, RPT)])
            plsc.subcore_barrier()
            fire_g(zs_hbm, 0, 0)
            fire_g(zs_hbm, 1, 1)
            wait_g(zs_hbm, 0)
            fire_s(0, 0)

            def pair(i, carry):
                ga = 2 * i + 1
                gb = 2 * i + 2
                wait_s(0)
                fire_g(zs_hbm, gb, 0)
                wait_g(zs_hbm, 1)
                fire_s(ga, 1)
                wait_s(1)
                fire_g(zs_hbm, gb + 1, 1)
                wait_g(zs_hbm, 0)
                fire_s(gb, 0)
                return carry

            lax.fori_loop(0, (NG - 2) // 2, pair, 0)
            wait_s(0)
            wait_g(zs_hbm, 1)
            fire_s(NG - 1, 1)
            wait_s(1)
            plsc.subcore_barrier()
            pltpu.sync_copy(acc.at[pl.ds(base, RPT)],
                            out_hbm.at[cid].at[pl.ds(base, RPT)])
            plsc.subcore_barrier()

        phase(zsa_hbm, outa_hbm)
        phase(zsb_hbm, outb_hbm)

    return prop2


def _deg_call():
    @functools.partial(
        pl.kernel,
        mesh=_sc_mesh(),
        compiler_params=pltpu.CompilerParams(use_tc_tiling_on_sc=False),
        out_type=jax.ShapeDtypeStruct((NCORE, NNP, 8), jnp.float32),
        scratch_types=[
            pltpu.VMEM((NCHUNK, KCH), jnp.int32),
            pltpu.VMEM((KCH, 8), jnp.float32),
            pltpu.VMEM_SHARED((NNP, 8), jnp.float32),
            pltpu.SemaphoreType.DMA,
        ],
    )
    def degk(dst_hbm, zeros_hbm, ones_hbm, out_hbm, dstv, ones_v, acc, ssem):
        cid = lax.axis_index("c")
        sid = lax.axis_index("s")
        wid = sid * NCORE + cid
        pltpu.sync_copy(dst_hbm.at[wid], dstv)
        pltpu.sync_copy(ones_hbm, ones_v)
        base = sid * RPT
        pltpu.sync_copy(zeros_hbm, acc.at[pl.ds(base, RPT)])
        plsc.subcore_barrier()

        for k in range(8):
            pltpu.async_copy(ones_v, acc.at[dstv.at[k]], ssem, add=True)

        def body(j, carry):
            pltpu.async_copy(ones_v, acc.at[dstv.at[j + 8]], ssem, add=True)
            pltpu.make_async_copy(ones_v, acc.at[dstv.at[j]], ssem).wait()
            return carry

        lax.fori_loop(0, NCHUNK - 8, body, 0)
        for k in range(8):
            pltpu.make_async_copy(ones_v, acc.at[dstv.at[k]], ssem).wait()
        plsc.subcore_barrier()
        pltpu.sync_copy(acc.at[pl.ds(base, RPT)],
                        out_hbm.at[cid].at[pl.ds(base, RPT)])

    return degk


def _ffn_l1(xp, W1f, c1, W2f, c2, cnt, Wc1p):
    R = 512

    def body(x_ref, w1_ref, c1_ref, w2_ref, c2_ref, cnt_ref, wc_ref,
             zsa_ref, zsb_ref, dinv_ref):
        h = jnp.dot(x_ref[...], w1_ref[...], preferred_element_type=jnp.float32)
        h = jnp.maximum(h + c1_ref[...], 0.0)
        h0 = jnp.dot(h, w2_ref[...],
                     preferred_element_type=jnp.float32) + c2_ref[...]
        deg = cnt_ref[0, :, 0:1] + cnt_ref[1, :, 0:1] + 1.0
        dinv = lax.rsqrt(deg)
        z = jnp.dot(h0, wc_ref[...], preferred_element_type=jnp.float32)
        zs = z * dinv
        zsa_ref[...] = zs[:, :32]
        zsb_ref[...] = zs[:, 32:]
        dinv_ref[...] = dinv

    return pl.pallas_call(
        body,
        grid=(NNP // R,),
        in_specs=[
            pl.BlockSpec((R, 128), lambda i: (i, 0)),
            pl.BlockSpec((128, 3840), lambda i: (0, 0)),
            pl.BlockSpec((1, 3840), lambda i: (0, 0)),
            pl.BlockSpec((3840, 128), lambda i: (0, 0)),
            pl.BlockSpec((1, 128), lambda i: (0, 0)),
            pl.BlockSpec((2, R, 8), lambda i: (0, i, 0)),
            pl.BlockSpec((128, 64), lambda i: (0, 0)),
        ],
        out_specs=[
            pl.BlockSpec((R, 32), lambda i: (i, 0)),
            pl.BlockSpec((R, 32), lambda i: (i, 0)),
            pl.BlockSpec((R, 1), lambda i: (i, 0)),
        ],
        out_shape=[
            jax.ShapeDtypeStruct((NNP, 32), jnp.float32),
            jax.ShapeDtypeStruct((NNP, 32), jnp.float32),
            jax.ShapeDtypeStruct((NNP, 1), jnp.float32),
        ],
    )(xp, W1f, c1, W2f, c2, cnt, Wc1p)


def _mid2(agga, aggb, zsa, zsb, dinv, Wc, bprev):
    R = 1024
    dpo = Wc.shape[1]

    def body(aa_ref, ab_ref, za_ref, zb_ref, d_ref, w_ref, b_ref, o_ref):
        sa = aa_ref[0] + aa_ref[1] - za_ref[...]
        sb = ab_ref[0] + ab_ref[1] - zb_ref[...]
        s = jnp.concatenate([sa, sb], axis=1)
        h = jnp.maximum(s * d_ref[...] + b_ref[...], 0.0)
        o_ref[...] = jnp.dot(h, w_ref[...],
                             preferred_element_type=jnp.float32) * d_ref[...]

    return pl.pallas_call(
        body,
        grid=(NNP // R,),
        in_specs=[
            pl.BlockSpec((2, R, 32), lambda i: (0, i, 0)),
            pl.BlockSpec((2, R, 32), lambda i: (0, i, 0)),
            pl.BlockSpec((R, 32), lambda i: (i, 0)),
            pl.BlockSpec((R, 32), lambda i: (i, 0)),
            pl.BlockSpec((R, 1), lambda i: (i, 0)),
            pl.BlockSpec((64, dpo), lambda i: (0, 0)),
            pl.BlockSpec((1, 64), lambda i: (0, 0)),
        ],
        out_specs=pl.BlockSpec((R, dpo), lambda i: (i, 0)),
        out_shape=jax.ShapeDtypeStruct((NNP, dpo), jnp.float32),
    )(agga, aggb, zsa, zsb, dinv, Wc, bprev)


def _mid(agg, zs_prev, dinv, Wc, bprev):
    R = 1024
    dpi = zs_prev.shape[1]
    dpo = Wc.shape[1]

    def body(a_ref, z_ref, d_ref, w_ref, b_ref, o_ref):
        s = a_ref[0] + a_ref[1] - z_ref[...]
        h = jnp.maximum(s * d_ref[...] + b_ref[...], 0.0)
        o_ref[...] = jnp.dot(h, w_ref[...],
                             preferred_element_type=jnp.float32) * d_ref[...]

    return pl.pallas_call(
        body,
        grid=(NNP // R,),
        in_specs=[
            pl.BlockSpec((2, R, dpi), lambda i: (0, i, 0)),
            pl.BlockSpec((R, dpi), lambda i: (i, 0)),
            pl.BlockSpec((R, 1), lambda i: (i, 0)),
            pl.BlockSpec((dpi, dpo), lambda i: (0, 0)),
            pl.BlockSpec((1, dpi), lambda i: (0, 0)),
        ],
        out_specs=pl.BlockSpec((R, dpo), lambda i: (i, 0)),
        out_shape=jax.ShapeDtypeStruct((NNP, dpo), jnp.float32),
    )(agg, zs_prev, dinv, Wc, bprev)


def _final_head(agg, zs_prev, dinv, bprev, Wfc3p, bfc2):
    R = 1024
    dpi = zs_prev.shape[1]

    def body(a_ref, z_ref, d_ref, b_ref, w_ref, bf_ref, o_ref):
        i = pl.program_id(0)
        s = a_ref[0] + a_ref[1] - z_ref[...]
        h = jnp.maximum(s * d_ref[...] + b_ref[...], 0.0)[:, :4]
        t = h[:, :, None] * w_ref[...]
        p = jnp.sum(jnp.sum(t, axis=0), axis=0)[None, :]

        @pl.when(i == 0)
        def _():
            o_ref[...] = p + bf_ref[...]

        @pl.when(i != 0)
        def _():
            o_ref[...] += p

    return pl.pallas_call(
        body,
        grid=(NNP // R,),
        in_specs=[
            pl.BlockSpec((2, R, dpi), lambda i: (0, i, 0)),
            pl.BlockSpec((R, dpi), lambda i: (i, 0)),
            pl.BlockSpec((R, 1), lambda i: (i, 0)),
            pl.BlockSpec((1, dpi), lambda i: (0, 0)),
            pl.BlockSpec((R, 4, 10), lambda i: (i, 0, 0)),
            pl.BlockSpec((1, 10), lambda i: (0, 0)),
        ],
        out_specs=pl.BlockSpec((1, 10), lambda i: (0, 0)),
        out_shape=jax.ShapeDtypeStruct((1, 10), jnp.float32),
    )(agg, zs_prev, dinv, bprev, Wfc3p, bfc2)


def kernel(x, edge_index, batch, W1, b1, g1, be1, rm1, rv1, W2, b2, g2, be2,
           rm2, rv2, Wc1, bc1, Wc2, bc2, Wc3, bc3, Wc4, bc4, Wc5, bc5, Wfc,
           bfc):
    eps = 1e-5
    xp = jnp.pad(x, ((0, NNP - NN), (0, 28)))
    s1 = g1 / jnp.sqrt(rv1 + eps)
    t1 = be1 - rm1 * s1
    W1f = jnp.pad(W1, ((0, 28), (0, 0))) * s1[None, :]
    c1 = (b1 * s1 + t1)[None, :]
    s2 = g2 / jnp.sqrt(rv2 + eps)
    t2 = be2 - rm2 * s2
    W2f = jnp.pad(W2 * s2[None, :], ((0, 0), (0, 28)))
    c2 = jnp.pad(b2 * s2 + t2, (0, 28))[None, :]
    Wc1p = jnp.pad(Wc1, ((0, 28), (0, 0)))
    Wc4p = jnp.pad(Wc4, ((0, 0), (0, 8)))
    Wc5p = jnp.pad(Wc5, ((0, 8), (0, 12)))
    bc4p = jnp.pad(bc4, (0, 8))[None, :]
    bc5p = jnp.pad(bc5, (0, 12))[None, :]
    pad_idx = NN + (jnp.arange(NEP - NE, dtype=jnp.int32) % (NNP - NN))
    src3 = jnp.concatenate([edge_index[0], pad_idx]).reshape(NW, NCHUNK, KCH)
    dst3 = jnp.concatenate([edge_index[1], pad_idx]).reshape(NW, NCHUNK, KCH)

    cnt = _deg_call()(dst3, jnp.zeros((RPT, 8), jnp.float32),
                      jnp.ones((KCH, 8), jnp.float32))
    zs1a, zs1b, dinv = _ffn_l1(xp, W1f, c1, W2f, c2, cnt, Wc1p)
    agg1a, agg1b = _prop2_call()(src3, dst3, zs1a, zs1b)
    zs2 = _mid2(agg1a, agg1b, zs1a, zs1b, dinv, Wc2, bc1[None, :])
    agg2 = _prop_call(32)(src3, dst3, zs2)
    zs3 = _mid(agg2, zs2, dinv, Wc3, bc2[None, :])
    agg3 = _prop_call(16)(src3, dst3, zs3)
    zs4 = _mid(agg3, zs3, dinv, Wc4p, bc3[None, :])
    agg4 = _prop_call(16)(src3, dst3, zs4)
    zs5 = _mid(agg4, zs4, dinv, Wc5p, bc4p)
    agg5 = _prop_call(16)(src3, dst3, zs5)
    Wfc3p = jnp.pad(Wfc.reshape(NN, 4, 10),
                    ((0, NNP - NN), (0, 0), (0, 0)))
    out = _final_head(agg5, zs5, dinv, bc5p, Wfc3p, bfc[None, :])
    return out
